# baseline (speedup 1.0000x reference)
import functools

import jax
import jax.numpy as jnp
from jax import lax
from jax.experimental import pallas as pl
from jax.experimental.pallas import tpu as pltpu
from jax.experimental.pallas import tpu_sc as plsc

VOCAB = 50257
EMBED = 768
B_TOTAL = 4 * 2048
NUM_WORKERS = 32
B_PER_W = B_TOTAL // NUM_WORKERS
CHUNK = 40
N_FULL = 6
TAIL = B_PER_W - N_FULL * CHUNK
NBUF = 4
PRIME = NBUF - 1
TAIL_BUF = (N_FULL + PRIME) % NBUF


@functools.partial(
    pl.kernel,
    mesh=plsc.VectorSubcoreMesh(core_axis_name="c", subcore_axis_name="s"),
    out_type=jax.ShapeDtypeStruct((B_TOTAL, EMBED), jnp.float32),
    scratch_types=[
        pltpu.VMEM((B_PER_W,), jnp.int32),
        pltpu.VMEM((NBUF, CHUNK, EMBED), jnp.float32),
        pltpu.SemaphoreType.DMA((NBUF,)),
        pltpu.SemaphoreType.DMA((NBUF,)),
        pltpu.SemaphoreType.DMA,
    ],
)
def _embed_lookup(table_hbm, idx_hbm, out_hbm, idx_v, rows_v, gsem, wsem, isem):
    c = lax.axis_index("c")
    s = lax.axis_index("s")
    wid = s * 2 + c
    base = wid * B_PER_W

    def gather_start(j, b):
        return pltpu.async_copy(
            table_hbm.at[idx_v.at[pl.ds(j * CHUNK, CHUNK)]],
            rows_v.at[b], gsem.at[b])

    def write_copy(j, b):
        return pltpu.make_async_copy(
            rows_v.at[b], out_hbm.at[pl.ds(base + j * CHUNK, CHUNK)],
            wsem.at[b])

    def tail_gather():
        return pltpu.async_copy(
            table_hbm.at[idx_v.at[pl.ds(N_FULL * CHUNK, TAIL)]],
            rows_v.at[TAIL_BUF, pl.ds(0, TAIL)], gsem.at[TAIL_BUF])

    head = PRIME * CHUNK
    pltpu.sync_copy(idx_hbm.at[pl.ds(base, head)], idx_v.at[pl.ds(0, head)])
    idx_rest = pltpu.async_copy(
        idx_hbm.at[pl.ds(base + head, B_PER_W - head)],
        idx_v.at[pl.ds(head, B_PER_W - head)], isem)
    for j in range(PRIME):
        gather_start(j, j)
    idx_rest.wait()

    def body(j, carry):
        b = lax.rem(j, NBUF)
        pltpu.make_async_copy(
            table_hbm.at[idx_v.at[pl.ds(j * CHUNK, CHUNK)]],
            rows_v.at[b], gsem.at[b]).wait()
        write_copy(j, b).start()
        nj = j + PRIME

        @pl.when(nj <= N_FULL)
        def _():
            @pl.when(j >= 1)
            def _():
                write_copy(j - 1, lax.rem(j - 1, NBUF)).wait()

            @pl.when(nj < N_FULL)
            def _():
                gather_start(nj, lax.rem(nj, NBUF))

            @pl.when(nj == N_FULL)
            def _():
                tail_gather()
        return carry

    lax.fori_loop(0, N_FULL, body, 0)
    pltpu.make_async_copy(
        table_hbm.at[idx_v.at[pl.ds(N_FULL * CHUNK, TAIL)]],
        rows_v.at[TAIL_BUF, pl.ds(0, TAIL)], gsem.at[TAIL_BUF]).wait()
    tail_write = pltpu.make_async_copy(
        rows_v.at[TAIL_BUF, pl.ds(0, TAIL)],
        out_hbm.at[pl.ds(base + N_FULL * CHUNK, TAIL)], wsem.at[TAIL_BUF])
    tail_write.start()
    for j in range(N_FULL - PRIME, N_FULL):
        write_copy(j, j % NBUF).wait()
    tail_write.wait()


def kernel(x, table):
    out = _embed_lookup(table, x.reshape(B_TOTAL).astype(jnp.int32))
    return out.reshape(x.shape[0], x.shape[1], EMBED)
